# Initial kernel scaffold; baseline (speedup 1.0000x reference)
import jax
import jax.numpy as jnp
from jax.experimental import pallas as pl


def kernel(x, enc1__0, enc1__1, enc1__2, enc2__0, enc2__1, enc2__2, enc3__0, enc3__1, enc3__2, bneck1__0, bneck1__1, bneck1__2, bneck2__0, bneck2__1, bneck2__2, up3__0, up3__1, dec3__0, dec3__1, dec3__2, up2__0, up2__1, dec2__0, dec2__1, dec2__2, up1__0, up1__1, dec1__0, dec1__1):
    raise NotImplementedError("write your pallas kernel here")


# speedup vs baseline: 1.1709x; 1.1709x over previous
import jax
import jax.numpy as jnp
from jax.experimental import pallas as pl
from jax.experimental.pallas import tpu as pltpu

_VMEM_CAP = 48 * 1024 * 1024
_INV_SQRT2 = 0.7071067811865476


def _gelu_erf(x):
    p = 0.3275911
    c1, c2, c3, c4, c5 = (0.254829592, -0.284496736, 1.421413741,
                          -1.453152027, 1.061405429)
    z = x * _INV_SQRT2
    az = jnp.abs(z)
    t = pl.reciprocal(1.0 + p * az, approx=True)
    poly = ((((c5 * t + c4) * t + c3) * t + c2) * t + c1) * t
    erf_z = jnp.sign(z) * (1.0 - poly * jnp.exp(-az * az))
    return 0.5 * x * (1.0 + erf_z)


def _pick_th(H, pref, even=False):
    for th in range(min(pref, H), 0, -1):
        if H % th == 0 and (not even or th % 2 == 0):
            return th
    return H


def _ilv(a, b, axis):
    st = jnp.stack([a, b], axis=axis + 1)
    shp = list(a.shape)
    shp[axis] *= 2
    return st.reshape(shp)


def _start_slab(x, n, r0, TH, W, buf, sem3, i, n_tiles, pad_w):
    def dst(row0, nrows):
        if pad_w:
            return buf.at[pl.ds(row0, nrows), pl.ds(16, W)]
        return buf.at[pl.ds(row0, nrows)]

    pltpu.make_async_copy(x.at[n, pl.ds(r0, TH)], dst(1, TH), sem3[0]).start()
    if n_tiles > 1:
        @pl.when(i > 0)
        def _():
            pltpu.make_async_copy(
                x.at[n, pl.ds(r0 - 1, 1)], dst(0, 1), sem3[1]).start()

        @pl.when(i < n_tiles - 1)
        def _():
            pltpu.make_async_copy(
                x.at[n, pl.ds(r0 + TH, 1)], dst(TH + 1, 1), sem3[2]).start()


def _wait_slab(x, n, TH, W, buf, sem3, i, n_tiles, pad_w):
    def dst(row0, nrows):
        if pad_w:
            return buf.at[pl.ds(row0, nrows), pl.ds(16, W)]
        return buf.at[pl.ds(row0, nrows)]

    pltpu.make_async_copy(x.at[n, pl.ds(0, TH)], dst(1, TH), sem3[0]).wait()
    if n_tiles > 1:
        @pl.when(i > 0)
        def _():
            pltpu.make_async_copy(x.at[n, pl.ds(0, 1)], dst(0, 1),
                                  sem3[1]).wait()

        @pl.when(i < n_tiles - 1)
        def _():
            pltpu.make_async_copy(x.at[n, pl.ds(0, 1)], dst(TH + 1, 1),
                                  sem3[2]).wait()


def _zero_pad(buf, TH, W, c, i, n_tiles):
    zc = jnp.zeros((TH + 2, 16, c), buf.dtype)
    buf[:, 0:16, :] = zc
    buf[:, W + 16:W + 32, :] = zc
    zr = jnp.zeros((1, W + 32, c), buf.dtype)
    if n_tiles > 1:
        @pl.when(i == 0)
        def _():
            buf[0:1] = zr

        @pl.when(i == n_tiles - 1)
        def _():
            buf[TH + 1:TH + 2] = zr
    else:
        buf[0:1] = zr
        buf[TH + 1:TH + 2] = zr


def _conv_acc(bufs, ws, cins, TH, W):
    acc = None
    for k, c in enumerate(cins):
        for kh in range(3):
            for kw in range(3):
                patch = bufs[k][kh:kh + TH,
                                kw + 15:kw + 15 + W, :].reshape(TH * W, c)
                if c == 1:
                    tap = ws[k][3 * kh + kw].reshape(1, -1)
                    d = patch.astype(jnp.float32) * tap
                else:
                    d = jnp.dot(patch, ws[k][3 * kh + kw],
                                preferred_element_type=jnp.float32)
                acc = d if acc is None else acc + d
    return acc


def _make_conv_body(TH, W, cins, Cout, n_tiles, do_gelu, do_pool):
    n_in = len(cins)

    def body(*refs):
        xs = refs[:n_in]
        ws = refs[n_in:2 * n_in]
        s_ref = refs[2 * n_in]
        t_ref = refs[2 * n_in + 1]
        o_ref = refs[2 * n_in + 2]
        bufs = refs[2 * n_in + 3:2 * n_in + 3 + n_in]
        sem = refs[2 * n_in + 3 + n_in]

        n = pl.program_id(0)
        i = pl.program_id(1)
        r0 = i * TH

        for k in range(n_in):
            _start_slab(xs[k], n, r0, TH, W, bufs[k],
                        (sem.at[k, 0], sem.at[k, 1], sem.at[k, 2]),
                        i, n_tiles, True)
        for k in range(n_in):
            _zero_pad(bufs[k], TH, W, cins[k], i, n_tiles)
        for k in range(n_in):
            _wait_slab(xs[k], n, TH, W, bufs[k],
                       (sem.at[k, 0], sem.at[k, 1], sem.at[k, 2]),
                       i, n_tiles, True)

        y = _conv_acc(bufs, ws, cins, TH, W)
        y = y * s_ref[...] + t_ref[...]
        if do_gelu:
            y = _gelu_erf(y)
        y = y.reshape(TH, W, Cout)
        if do_pool:
            y = y.reshape(TH // 2, 2, W, Cout).max(axis=1)
            y = y.reshape(TH // 2, W // 2, 2, Cout).max(axis=2)
        o_ref[0] = y.astype(o_ref.dtype)

    return body


def _conv_stage(xs, ws_raw, scale, shift, *, gelu, pool, th_pref=16):
    N, H, W, _ = xs[0].shape
    cins = tuple(int(a.shape[-1]) for a in xs)
    Cout = int(ws_raw[0].shape[-1])
    n_in = len(xs)
    TH = _pick_th(H, th_pref, even=pool)
    n_tiles = H // TH
    Ho, Wo = (H // 2, W // 2) if pool else (H, W)
    THo = TH // 2 if pool else TH

    wfs, w_specs = [], []
    for w, c in zip(ws_raw, cins):
        if c == 1:
            wfs.append(w.reshape(9, Cout).astype(jnp.float32))
            w_specs.append(pl.BlockSpec((9, Cout), lambda n, i: (0, 0)))
        else:
            wfs.append(w.astype(jnp.bfloat16).reshape(9, c, Cout))
            w_specs.append(
                pl.BlockSpec((9, c, Cout), lambda n, i: (0, 0, 0)))

    return pl.pallas_call(
        _make_conv_body(TH, W, cins, Cout, n_tiles, gelu, pool),
        out_shape=jax.ShapeDtypeStruct((N, Ho, Wo, Cout), jnp.bfloat16),
        grid=(N, n_tiles),
        in_specs=([pl.BlockSpec(memory_space=pl.ANY)] * n_in + w_specs
                  + [pl.BlockSpec((1, Cout), lambda n, i: (0, 0)),
                     pl.BlockSpec((1, Cout), lambda n, i: (0, 0))]),
        out_specs=pl.BlockSpec((1, THo, Wo, Cout), lambda n, i: (n, i, 0, 0)),
        scratch_shapes=([pltpu.VMEM((TH + 2, W + 32, c), x.dtype)
                         for c, x in zip(cins, xs)]
                        + [pltpu.SemaphoreType.DMA((n_in, 3))]),
        compiler_params=pltpu.CompilerParams(
            dimension_semantics=("parallel", "parallel"),
            vmem_limit_bytes=_VMEM_CAP),
    )(*xs, *wfs, jnp.asarray(scale, jnp.float32),
      jnp.asarray(shift, jnp.float32))


def _make_dec_body(THd, Wb, Cb, Wd, Cu, Ce, Cout, n_tiles):
    THb = THd // 2

    def body(bt_hbm, e_hbm, wu, bu, wcu, wce, s_ref, t_ref, o_ref,
             btbuf, ubuf, ebuf, sem):
        n = pl.program_id(0)
        i = pl.program_id(1)
        r0 = i * THd
        rb0 = i * THb

        _start_slab(bt_hbm, n, rb0, THb, Wb, btbuf,
                    (sem.at[0], sem.at[1], sem.at[2]), i, n_tiles, False)
        _start_slab(e_hbm, n, r0, THd, Wd, ebuf,
                    (sem.at[3], sem.at[4], sem.at[5]), i, n_tiles, True)
        _zero_pad(ebuf, THd, Wd, Ce, i, n_tiles)

        zc = jnp.zeros((THd + 2, 16, Cu), ubuf.dtype)
        ubuf[:, 0:16, :] = zc
        ubuf[:, Wd + 16:Wd + 32, :] = zc

        _wait_slab(bt_hbm, n, THb, Wb, btbuf,
                   (sem.at[0], sem.at[1], sem.at[2]), i, n_tiles, False)

        bias = bu[...]

        def up_row(src, dh):
            a = _gelu_erf(jnp.dot(src, wu[2 * dh + 0],
                                  preferred_element_type=jnp.float32) + bias)
            b = _gelu_erf(jnp.dot(src, wu[2 * dh + 1],
                                  preferred_element_type=jnp.float32) + bias)
            return _ilv(a.astype(ubuf.dtype).reshape(-1, Wb, Cu),
                        b.astype(ubuf.dtype).reshape(-1, Wb, Cu), 1)

        xm = btbuf[1:THb + 1].reshape(THb * Wb, Cb)
        top = up_row(xm, 0)
        bot = up_row(xm, 1)
        ubuf[1:THd + 1, 16:Wd + 16] = _ilv(top, bot, 0)

        zrow = jnp.zeros((1, Wd + 32, Cu), ubuf.dtype)
        if n_tiles > 1:
            @pl.when(i > 0)
            def _():
                ubuf[0:1, 16:Wd + 16] = up_row(btbuf[0], 1)

            @pl.when(i == 0)
            def _():
                ubuf[0:1] = zrow

            @pl.when(i < n_tiles - 1)
            def _():
                ubuf[THd + 1:THd + 2, 16:Wd + 16] = up_row(btbuf[THb + 1], 0)

            @pl.when(i == n_tiles - 1)
            def _():
                ubuf[THd + 1:THd + 2] = zrow
        else:
            ubuf[0:1] = zrow
            ubuf[THd + 1:THd + 2] = zrow

        _wait_slab(e_hbm, n, THd, Wd, ebuf,
                   (sem.at[3], sem.at[4], sem.at[5]), i, n_tiles, True)

        y = _conv_acc((ubuf, ebuf), (wcu, wce), (Cu, Ce), THd, Wd)
        y = y * s_ref[...] + t_ref[...]
        o_ref[0] = y.reshape(THd, Wd, Cout).astype(o_ref.dtype)

    return body


def _dec_stage(xin, eskip, wu_raw, bu, wc_raw, scale, shift, *, thd_pref=16):
    N, Hb, Wb, Cb = xin.shape
    _, Hd, Wd, Ce = eskip.shape
    Cu = int(wu_raw.shape[-1])
    Cout = int(wc_raw.shape[-1])
    THd = _pick_th(Hd, thd_pref, even=True)
    n_tiles = Hd // THd

    wu = wu_raw.astype(jnp.bfloat16).reshape(4, Cb, Cu)
    wcu = wc_raw[:, :, :Cu, :].astype(jnp.bfloat16).reshape(9, Cu, Cout)
    wce = wc_raw[:, :, Cu:, :].astype(jnp.bfloat16).reshape(9, Ce, Cout)

    return pl.pallas_call(
        _make_dec_body(THd, Wb, Cb, Wd, Cu, Ce, Cout, n_tiles),
        out_shape=jax.ShapeDtypeStruct((N, Hd, Wd, Cout), jnp.bfloat16),
        grid=(N, n_tiles),
        in_specs=[
            pl.BlockSpec(memory_space=pl.ANY),
            pl.BlockSpec(memory_space=pl.ANY),
            pl.BlockSpec((4, Cb, Cu), lambda n, i: (0, 0, 0)),
            pl.BlockSpec((1, Cu), lambda n, i: (0, 0)),
            pl.BlockSpec((9, Cu, Cout), lambda n, i: (0, 0, 0)),
            pl.BlockSpec((9, Ce, Cout), lambda n, i: (0, 0, 0)),
            pl.BlockSpec((1, Cout), lambda n, i: (0, 0)),
            pl.BlockSpec((1, Cout), lambda n, i: (0, 0)),
        ],
        out_specs=pl.BlockSpec((1, THd, Wd, Cout), lambda n, i: (n, i, 0, 0)),
        scratch_shapes=[
            pltpu.VMEM((THd // 2 + 2, Wb, Cb), jnp.bfloat16),
            pltpu.VMEM((THd + 2, Wd + 32, Cu), jnp.bfloat16),
            pltpu.VMEM((THd + 2, Wd + 32, Ce), jnp.bfloat16),
            pltpu.SemaphoreType.DMA((6,)),
        ],
        compiler_params=pltpu.CompilerParams(
            dimension_semantics=("parallel", "parallel"),
            vmem_limit_bytes=_VMEM_CAP),
    )(xin, eskip, wu, jnp.asarray(bu, jnp.float32), wcu, wce,
      jnp.asarray(scale, jnp.float32), jnp.asarray(shift, jnp.float32))


def _make_head_body(THp, W2, C2, Cu):
    def body(x_ref, wu_ref, bu_ref, wp_ref, bp_ref, o_ref):
        xm = x_ref[0].reshape(THp * W2, C2)
        bias = bu_ref[...]
        wpv = wp_ref[...]
        rows = []
        for dh in range(2):
            cols = []
            for dw in range(2):
                y = _gelu_erf(jnp.dot(xm, wu_ref[2 * dh + dw],
                                      preferred_element_type=jnp.float32)
                              + bias)
                p = jnp.sum(y * wpv, axis=-1, keepdims=True) + bp_ref[...]
                cols.append(p.reshape(THp, W2))
            rows.append(_ilv(cols[0], cols[1], 1))
        out = _ilv(rows[0], rows[1], 0)
        o_ref[0, 0] = out.astype(o_ref.dtype)

    return body


def _head_stage(x, wu_raw, bu, wp_raw, bp, *, thp_pref=16):
    N, H2, W2, C2 = x.shape
    Cu = int(wu_raw.shape[-1])
    THp = _pick_th(H2, thp_pref)
    n_tiles = H2 // THp

    wu = wu_raw.astype(jnp.bfloat16).reshape(4, C2, Cu)
    wp = wp_raw.reshape(1, Cu).astype(jnp.float32)

    return pl.pallas_call(
        _make_head_body(THp, W2, C2, Cu),
        out_shape=jax.ShapeDtypeStruct((N, 1, 2 * H2, 2 * W2), jnp.float32),
        grid=(N, n_tiles),
        in_specs=[
            pl.BlockSpec((1, THp, W2, C2), lambda n, i: (n, i, 0, 0)),
            pl.BlockSpec((4, C2, Cu), lambda n, i: (0, 0, 0)),
            pl.BlockSpec((1, Cu), lambda n, i: (0, 0)),
            pl.BlockSpec((1, Cu), lambda n, i: (0, 0)),
            pl.BlockSpec((1, 1), lambda n, i: (0, 0)),
        ],
        out_specs=pl.BlockSpec((1, 1, 2 * THp, 2 * W2),
                               lambda n, i: (n, 0, i, 0)),
        compiler_params=pltpu.CompilerParams(
            dimension_semantics=("parallel", "parallel"),
            vmem_limit_bytes=_VMEM_CAP),
    )(x, wu, jnp.asarray(bu, jnp.float32), wp, jnp.asarray(bp, jnp.float32))


def kernel(x,
           enc1__0, enc1__1, enc1__2,
           enc2__0, enc2__1, enc2__2,
           enc3__0, enc3__1, enc3__2,
           bneck1__0, bneck1__1, bneck1__2,
           bneck2__0, bneck2__1, bneck2__2,
           up3__0, up3__1,
           dec3__0, dec3__1, dec3__2,
           up2__0, up2__1,
           dec2__0, dec2__1, dec2__2,
           up1__0, up1__1,
           dec1__0, dec1__1):
    N = x.shape[0]
    F, T = x.shape[-2:]
    xh = x.reshape(N, F, T, 1)

    e1 = _conv_stage([xh], [enc1__0], enc1__1, enc1__2, gelu=True, pool=True)
    e2 = _conv_stage([e1], [enc2__0], enc2__1, enc2__2, gelu=True, pool=True)
    e3 = _conv_stage([e2], [enc3__0], enc3__1, enc3__2, gelu=True, pool=True)

    bt = _conv_stage([e3], [bneck1__0], bneck1__1, bneck1__2,
                     gelu=True, pool=False)
    bt = _conv_stage([bt], [bneck2__0], bneck2__1, bneck2__2,
                     gelu=True, pool=False)

    d3 = _dec_stage(bt, e2, up3__0, up3__1, dec3__0, dec3__1, dec3__2)
    d2 = _dec_stage(d3, e1, up2__0, up2__1, dec2__0, dec2__1, dec2__2)
    return _head_stage(d2, up1__0, up1__1, dec1__0, dec1__1)


# speedup vs baseline: 2.0429x; 1.7447x over previous
import jax
import jax.numpy as jnp
from jax.experimental import pallas as pl
from jax.experimental.pallas import tpu as pltpu

_VMEM_CAP = 48 * 1024 * 1024
_INV_SQRT2 = 0.7071067811865476


def _gelu_erf(x):
    c = 0.7978845608028654
    return 0.5 * x * (1.0 + jnp.tanh(c * x * (1.0 + 0.044715 * x * x)))


def _pick_th(H, pref, even=False):
    for th in range(min(pref, H), 0, -1):
        if H % th == 0 and (not even or th % 2 == 0):
            return th
    return H


def _ilv(a, b, axis):
    st = jnp.stack([a, b], axis=axis + 1)
    shp = list(a.shape)
    shp[axis] *= 2
    return st.reshape(shp)


def _start_slab(x, n, r0, TH, W, buf, sem3, i, n_tiles, pad_w):
    def dst(row0, nrows):
        if pad_w:
            return buf.at[pl.ds(row0, nrows), pl.ds(16, W)]
        return buf.at[pl.ds(row0, nrows)]

    pltpu.make_async_copy(x.at[n, pl.ds(r0, TH)], dst(1, TH), sem3[0]).start()
    if n_tiles > 1:
        @pl.when(i > 0)
        def _():
            pltpu.make_async_copy(
                x.at[n, pl.ds(r0 - 1, 1)], dst(0, 1), sem3[1]).start()

        @pl.when(i < n_tiles - 1)
        def _():
            pltpu.make_async_copy(
                x.at[n, pl.ds(r0 + TH, 1)], dst(TH + 1, 1), sem3[2]).start()


def _wait_slab(x, n, TH, W, buf, sem3, i, n_tiles, pad_w):
    def dst(row0, nrows):
        if pad_w:
            return buf.at[pl.ds(row0, nrows), pl.ds(16, W)]
        return buf.at[pl.ds(row0, nrows)]

    pltpu.make_async_copy(x.at[n, pl.ds(0, TH)], dst(1, TH), sem3[0]).wait()
    if n_tiles > 1:
        @pl.when(i > 0)
        def _():
            pltpu.make_async_copy(x.at[n, pl.ds(0, 1)], dst(0, 1),
                                  sem3[1]).wait()

        @pl.when(i < n_tiles - 1)
        def _():
            pltpu.make_async_copy(x.at[n, pl.ds(0, 1)], dst(TH + 1, 1),
                                  sem3[2]).wait()


def _zero_pad(buf, TH, W, c, i, n_tiles):
    zc = jnp.zeros((TH + 2, 16, c), buf.dtype)
    buf[:, 0:16, :] = zc
    buf[:, W + 16:W + 32, :] = zc
    zr = jnp.zeros((1, W + 32, c), buf.dtype)
    if n_tiles > 1:
        @pl.when(i == 0)
        def _():
            buf[0:1] = zr

        @pl.when(i == n_tiles - 1)
        def _():
            buf[TH + 1:TH + 2] = zr
    else:
        buf[0:1] = zr
        buf[TH + 1:TH + 2] = zr


def _conv_acc(bufs, wcat, cins, TH, W):
    if cins == (1,):
        cat = jnp.concatenate(
            [bufs[0][kh:kh + TH, kw + 15:kw + 15 + W, :].astype(jnp.bfloat16)
             for kh in range(3) for kw in range(3)], axis=-1)
        return jnp.dot(cat.reshape(TH * W, 9), wcat[0],
                       preferred_element_type=jnp.float32)
    cat = jnp.concatenate(
        [bufs[k][:, kw + 15:kw + 15 + W, :]
         for k in range(len(cins)) for kw in range(3)], axis=-1)
    K = 3 * sum(cins)
    acc = None
    for kh in range(3):
        patch = cat[kh:kh + TH].reshape(TH * W, K)
        d = jnp.dot(patch, wcat[kh], preferred_element_type=jnp.float32)
        acc = d if acc is None else acc + d
    return acc


def _make_conv_body(TH, W, cins, Cout, n_tiles, do_gelu, do_pool):
    n_in = len(cins)

    def body(*refs):
        xs = refs[:n_in]
        wcat = refs[n_in]
        s_ref = refs[n_in + 1]
        t_ref = refs[n_in + 2]
        o_ref = refs[n_in + 3]
        bufs = refs[n_in + 4:2 * n_in + 4]
        sem = refs[2 * n_in + 4]

        n = pl.program_id(0)
        i = pl.program_id(1)
        r0 = i * TH

        for k in range(n_in):
            _start_slab(xs[k], n, r0, TH, W, bufs[k],
                        (sem.at[k, 0], sem.at[k, 1], sem.at[k, 2]),
                        i, n_tiles, True)
        for k in range(n_in):
            _zero_pad(bufs[k], TH, W, cins[k], i, n_tiles)
        for k in range(n_in):
            _wait_slab(xs[k], n, TH, W, bufs[k],
                       (sem.at[k, 0], sem.at[k, 1], sem.at[k, 2]),
                       i, n_tiles, True)

        y = _conv_acc(bufs, wcat, cins, TH, W)
        y = y * s_ref[...] + t_ref[...]
        if do_gelu:
            y = _gelu_erf(y)
        y = y.reshape(TH, W, Cout)
        if do_pool:
            y = y.reshape(TH // 2, 2, W, Cout).max(axis=1)
            y = y.reshape(TH // 2, W // 2, 2, Cout).max(axis=2)
        o_ref[0] = y.astype(o_ref.dtype)

    return body


def _conv_stage(xs, ws_raw, scale, shift, *, gelu, pool, th_pref=16):
    N, H, W, _ = xs[0].shape
    cins = tuple(int(a.shape[-1]) for a in xs)
    Cout = int(ws_raw[0].shape[-1])
    n_in = len(xs)
    TH = _pick_th(H, th_pref, even=pool)
    n_tiles = H // TH
    Ho, Wo = (H // 2, W // 2) if pool else (H, W)
    THo = TH // 2 if pool else TH

    if cins == (1,):
        wcat = ws_raw[0].reshape(9, Cout).astype(jnp.bfloat16)[None]
        w_spec = pl.BlockSpec((1, 9, Cout), lambda n, i: (0, 0, 0))
    else:
        K = 3 * sum(cins)
        wcat = jnp.stack([
            jnp.concatenate([ws_raw[k].astype(jnp.bfloat16)[kh, kw]
                             for k in range(n_in) for kw in range(3)], axis=0)
            for kh in range(3)])
        w_spec = pl.BlockSpec((3, K, Cout), lambda n, i: (0, 0, 0))

    return pl.pallas_call(
        _make_conv_body(TH, W, cins, Cout, n_tiles, gelu, pool),
        out_shape=jax.ShapeDtypeStruct((N, Ho, Wo, Cout), jnp.bfloat16),
        grid=(N, n_tiles),
        in_specs=([pl.BlockSpec(memory_space=pl.ANY)] * n_in + [w_spec]
                  + [pl.BlockSpec((1, Cout), lambda n, i: (0, 0)),
                     pl.BlockSpec((1, Cout), lambda n, i: (0, 0))]),
        out_specs=pl.BlockSpec((1, THo, Wo, Cout), lambda n, i: (n, i, 0, 0)),
        scratch_shapes=([pltpu.VMEM((TH + 2, W + 32, c), x.dtype)
                         for c, x in zip(cins, xs)]
                        + [pltpu.SemaphoreType.DMA((n_in, 3))]),
        compiler_params=pltpu.CompilerParams(
            dimension_semantics=("parallel", "parallel"),
            vmem_limit_bytes=_VMEM_CAP),
    )(*xs, wcat, jnp.asarray(scale, jnp.float32),
      jnp.asarray(shift, jnp.float32))


def _make_dec_body(THd, Wb, Cb, Wd, Cu, Ce, Cout, n_tiles):
    THb = THd // 2

    def body(bt_hbm, e_hbm, wu, bu, wcat, s_ref, t_ref, o_ref,
             btbuf, ubuf, ebuf, sem):
        n = pl.program_id(0)
        i = pl.program_id(1)
        r0 = i * THd
        rb0 = i * THb

        _start_slab(bt_hbm, n, rb0, THb, Wb, btbuf,
                    (sem.at[0], sem.at[1], sem.at[2]), i, n_tiles, False)
        _start_slab(e_hbm, n, r0, THd, Wd, ebuf,
                    (sem.at[3], sem.at[4], sem.at[5]), i, n_tiles, True)
        _zero_pad(ebuf, THd, Wd, Ce, i, n_tiles)

        zc = jnp.zeros((THd + 2, 16, Cu), ubuf.dtype)
        ubuf[:, 0:16, :] = zc
        ubuf[:, Wd + 16:Wd + 32, :] = zc

        _wait_slab(bt_hbm, n, THb, Wb, btbuf,
                   (sem.at[0], sem.at[1], sem.at[2]), i, n_tiles, False)

        bias = bu[...]

        def up_row(src, dh):
            a = _gelu_erf(jnp.dot(src, wu[2 * dh + 0],
                                  preferred_element_type=jnp.float32) + bias)
            b = _gelu_erf(jnp.dot(src, wu[2 * dh + 1],
                                  preferred_element_type=jnp.float32) + bias)
            return _ilv(a.astype(ubuf.dtype).reshape(-1, Wb, Cu),
                        b.astype(ubuf.dtype).reshape(-1, Wb, Cu), 1)

        xm = btbuf[1:THb + 1].reshape(THb * Wb, Cb)
        top = up_row(xm, 0)
        bot = up_row(xm, 1)
        ubuf[1:THd + 1, 16:Wd + 16] = _ilv(top, bot, 0)

        zrow = jnp.zeros((1, Wd + 32, Cu), ubuf.dtype)
        if n_tiles > 1:
            @pl.when(i > 0)
            def _():
                ubuf[0:1, 16:Wd + 16] = up_row(btbuf[0], 1)

            @pl.when(i == 0)
            def _():
                ubuf[0:1] = zrow

            @pl.when(i < n_tiles - 1)
            def _():
                ubuf[THd + 1:THd + 2, 16:Wd + 16] = up_row(btbuf[THb + 1], 0)

            @pl.when(i == n_tiles - 1)
            def _():
                ubuf[THd + 1:THd + 2] = zrow
        else:
            ubuf[0:1] = zrow
            ubuf[THd + 1:THd + 2] = zrow

        _wait_slab(e_hbm, n, THd, Wd, ebuf,
                   (sem.at[3], sem.at[4], sem.at[5]), i, n_tiles, True)

        y = _conv_acc((ubuf, ebuf), wcat, (Cu, Ce), THd, Wd)
        y = y * s_ref[...] + t_ref[...]
        o_ref[0] = y.reshape(THd, Wd, Cout).astype(o_ref.dtype)

    return body


def _dec_stage(xin, eskip, wu_raw, bu, wc_raw, scale, shift, *, thd_pref=16):
    N, Hb, Wb, Cb = xin.shape
    _, Hd, Wd, Ce = eskip.shape
    Cu = int(wu_raw.shape[-1])
    Cout = int(wc_raw.shape[-1])
    THd = _pick_th(Hd, thd_pref, even=True)
    n_tiles = Hd // THd

    wu = wu_raw.astype(jnp.bfloat16).reshape(4, Cb, Cu)
    wcb = wc_raw.astype(jnp.bfloat16)
    wcat = jnp.stack([
        jnp.concatenate([wcb[kh, kw, :Cu] for kw in range(3)]
                        + [wcb[kh, kw, Cu:] for kw in range(3)], axis=0)
        for kh in range(3)])

    return pl.pallas_call(
        _make_dec_body(THd, Wb, Cb, Wd, Cu, Ce, Cout, n_tiles),
        out_shape=jax.ShapeDtypeStruct((N, Hd, Wd, Cout), jnp.bfloat16),
        grid=(N, n_tiles),
        in_specs=[
            pl.BlockSpec(memory_space=pl.ANY),
            pl.BlockSpec(memory_space=pl.ANY),
            pl.BlockSpec((4, Cb, Cu), lambda n, i: (0, 0, 0)),
            pl.BlockSpec((1, Cu), lambda n, i: (0, 0)),
            pl.BlockSpec((3, 3 * (Cu + Ce), Cout), lambda n, i: (0, 0, 0)),
            pl.BlockSpec((1, Cout), lambda n, i: (0, 0)),
            pl.BlockSpec((1, Cout), lambda n, i: (0, 0)),
        ],
        out_specs=pl.BlockSpec((1, THd, Wd, Cout), lambda n, i: (n, i, 0, 0)),
        scratch_shapes=[
            pltpu.VMEM((THd // 2 + 2, Wb, Cb), jnp.bfloat16),
            pltpu.VMEM((THd + 2, Wd + 32, Cu), jnp.bfloat16),
            pltpu.VMEM((THd + 2, Wd + 32, Ce), jnp.bfloat16),
            pltpu.SemaphoreType.DMA((6,)),
        ],
        compiler_params=pltpu.CompilerParams(
            dimension_semantics=("parallel", "parallel"),
            vmem_limit_bytes=_VMEM_CAP),
    )(xin, eskip, wu, jnp.asarray(bu, jnp.float32), wcat,
      jnp.asarray(scale, jnp.float32), jnp.asarray(shift, jnp.float32))


def _make_head_body(THp, W2, C2, Cu):
    def body(x_ref, wu_ref, bu_ref, wp_ref, bp_ref, o_ref):
        xm = x_ref[0].reshape(THp * W2, C2)
        bias = bu_ref[...]
        wpv = wp_ref[...]
        for dh in range(2):
            for dw in range(2):
                y = _gelu_erf(jnp.dot(xm, wu_ref[2 * dh + dw],
                                      preferred_element_type=jnp.float32)
                              + bias)
                p = jnp.sum(y * wpv, axis=-1, keepdims=True) + bp_ref[...]
                o_ref[0, 2 * dh + dw] = p.reshape(THp, W2).astype(o_ref.dtype)

    return body


def _head_stage(x, wu_raw, bu, wp_raw, bp, *, thp_pref=16):
    N, H2, W2, C2 = x.shape
    Cu = int(wu_raw.shape[-1])
    THp = _pick_th(H2, thp_pref)
    n_tiles = H2 // THp

    wu = wu_raw.astype(jnp.bfloat16).reshape(4, C2, Cu)
    wp = wp_raw.reshape(1, Cu).astype(jnp.float32)

    planar = pl.pallas_call(
        _make_head_body(THp, W2, C2, Cu),
        out_shape=jax.ShapeDtypeStruct((N, 4, H2, W2), jnp.float32),
        grid=(N, n_tiles),
        in_specs=[
            pl.BlockSpec((1, THp, W2, C2), lambda n, i: (n, i, 0, 0)),
            pl.BlockSpec((4, C2, Cu), lambda n, i: (0, 0, 0)),
            pl.BlockSpec((1, Cu), lambda n, i: (0, 0)),
            pl.BlockSpec((1, Cu), lambda n, i: (0, 0)),
            pl.BlockSpec((1, 1), lambda n, i: (0, 0)),
        ],
        out_specs=pl.BlockSpec((1, 4, THp, W2), lambda n, i: (n, 0, i, 0)),
        compiler_params=pltpu.CompilerParams(
            dimension_semantics=("parallel", "parallel"),
            vmem_limit_bytes=_VMEM_CAP),
    )(x, wu, jnp.asarray(bu, jnp.float32), wp, jnp.asarray(bp, jnp.float32))
    out = planar.reshape(N, 2, 2, H2, W2).transpose(0, 3, 1, 4, 2)
    return out.reshape(N, 1, 2 * H2, 2 * W2)


def kernel(x,
           enc1__0, enc1__1, enc1__2,
           enc2__0, enc2__1, enc2__2,
           enc3__0, enc3__1, enc3__2,
           bneck1__0, bneck1__1, bneck1__2,
           bneck2__0, bneck2__1, bneck2__2,
           up3__0, up3__1,
           dec3__0, dec3__1, dec3__2,
           up2__0, up2__1,
           dec2__0, dec2__1, dec2__2,
           up1__0, up1__1,
           dec1__0, dec1__1):
    N = x.shape[0]
    F, T = x.shape[-2:]
    xh = x.reshape(N, F, T, 1)

    e1 = _conv_stage([xh], [enc1__0], enc1__1, enc1__2, gelu=True, pool=True)
    e2 = _conv_stage([e1], [enc2__0], enc2__1, enc2__2, gelu=True, pool=True)
    e3 = _conv_stage([e2], [enc3__0], enc3__1, enc3__2, gelu=True, pool=True)

    bt = _conv_stage([e3], [bneck1__0], bneck1__1, bneck1__2,
                     gelu=True, pool=False)
    bt = _conv_stage([bt], [bneck2__0], bneck2__1, bneck2__2,
                     gelu=True, pool=False)

    d3 = _dec_stage(bt, e2, up3__0, up3__1, dec3__0, dec3__1, dec3__2)
    d2 = _dec_stage(d3, e1, up2__0, up2__1, dec2__0, dec2__1, dec2__2)
    return _head_stage(d2, up1__0, up1__1, dec1__0, dec1__1)


# speedup vs baseline: 2.1862x; 1.0701x over previous
import jax
import jax.numpy as jnp
from jax.experimental import pallas as pl
from jax.experimental.pallas import tpu as pltpu

_VMEM_CAP = 48 * 1024 * 1024
_INV_SQRT2 = 0.7071067811865476


def _gelu_erf(x):
    c = 0.7978845608028654
    return 0.5 * x * (1.0 + jnp.tanh(c * x * (1.0 + 0.044715 * x * x)))


def _pick_th(H, pref, even=False):
    for th in range(min(pref, H), 0, -1):
        if H % th == 0 and (not even or th % 2 == 0):
            return th
    return H


def _ilv(a, b, axis):
    st = jnp.stack([a, b], axis=axis + 1)
    shp = list(a.shape)
    shp[axis] *= 2
    return st.reshape(shp)


def _start_slab(x, n, r0, TH, W, buf, sem3, i, n_tiles, pad_w):
    def dst(row0, nrows):
        if pad_w:
            return buf.at[pl.ds(row0, nrows), pl.ds(16, W)]
        return buf.at[pl.ds(row0, nrows)]

    pltpu.make_async_copy(x.at[n, pl.ds(r0, TH)], dst(1, TH), sem3[0]).start()
    if n_tiles > 1:
        @pl.when(i > 0)
        def _():
            pltpu.make_async_copy(
                x.at[n, pl.ds(r0 - 1, 1)], dst(0, 1), sem3[1]).start()

        @pl.when(i < n_tiles - 1)
        def _():
            pltpu.make_async_copy(
                x.at[n, pl.ds(r0 + TH, 1)], dst(TH + 1, 1), sem3[2]).start()


def _wait_slab(x, n, TH, W, buf, sem3, i, n_tiles, pad_w):
    def dst(row0, nrows):
        if pad_w:
            return buf.at[pl.ds(row0, nrows), pl.ds(16, W)]
        return buf.at[pl.ds(row0, nrows)]

    pltpu.make_async_copy(x.at[n, pl.ds(0, TH)], dst(1, TH), sem3[0]).wait()
    if n_tiles > 1:
        @pl.when(i > 0)
        def _():
            pltpu.make_async_copy(x.at[n, pl.ds(0, 1)], dst(0, 1),
                                  sem3[1]).wait()

        @pl.when(i < n_tiles - 1)
        def _():
            pltpu.make_async_copy(x.at[n, pl.ds(0, 1)], dst(TH + 1, 1),
                                  sem3[2]).wait()


def _zero_pad(buf, TH, W, c, i, n_tiles):
    zc = jnp.zeros((TH + 2, 16, c), buf.dtype)
    buf[:, 0:16, :] = zc
    buf[:, W + 16:W + 32, :] = zc
    zr = jnp.zeros((1, W + 32, c), buf.dtype)
    if n_tiles > 1:
        @pl.when(i == 0)
        def _():
            buf[0:1] = zr

        @pl.when(i == n_tiles - 1)
        def _():
            buf[TH + 1:TH + 2] = zr
    else:
        buf[0:1] = zr
        buf[TH + 1:TH + 2] = zr


def _conv_acc(bufs, wcat, cins, TH, W):
    if cins == (1,):
        cat = jnp.concatenate(
            [bufs[0][kh:kh + TH, kw + 15:kw + 15 + W, :].astype(jnp.bfloat16)
             for kh in range(3) for kw in range(3)], axis=-1)
        return jnp.dot(cat.reshape(TH * W, 9), wcat[0],
                       preferred_element_type=jnp.float32)
    cat = jnp.concatenate(
        [bufs[k][:, kw + 15:kw + 15 + W, :]
         for k in range(len(cins)) for kw in range(3)], axis=-1)
    K = 3 * sum(cins)
    acc = None
    for kh in range(3):
        patch = cat[kh:kh + TH].reshape(TH * W, K)
        d = jnp.dot(patch, wcat[kh], preferred_element_type=jnp.float32)
        acc = d if acc is None else acc + d
    return acc


def _make_conv_body(TH, W, cins, Cout, n_tiles, do_gelu, do_pool):
    n_in = len(cins)

    def body(*refs):
        xs = refs[:n_in]
        wcat = refs[n_in]
        s_ref = refs[n_in + 1]
        t_ref = refs[n_in + 2]
        o_ref = refs[n_in + 3]
        bufs = refs[n_in + 4:2 * n_in + 4]
        sem = refs[2 * n_in + 4]

        n = pl.program_id(0)
        i = pl.program_id(1)
        r0 = i * TH

        for k in range(n_in):
            _start_slab(xs[k], n, r0, TH, W, bufs[k],
                        (sem.at[k, 0], sem.at[k, 1], sem.at[k, 2]),
                        i, n_tiles, True)
        for k in range(n_in):
            _zero_pad(bufs[k], TH, W, cins[k], i, n_tiles)
        for k in range(n_in):
            _wait_slab(xs[k], n, TH, W, bufs[k],
                       (sem.at[k, 0], sem.at[k, 1], sem.at[k, 2]),
                       i, n_tiles, True)

        y = _conv_acc(bufs, wcat, cins, TH, W)
        y = y * s_ref[...] + t_ref[...]
        if do_gelu:
            y = _gelu_erf(y)
        y = y.reshape(TH, W, Cout)
        if do_pool:
            y = y.reshape(TH // 2, 2, W, Cout).max(axis=1)
            y = y.reshape(TH // 2, W // 2, 2, Cout).max(axis=2)
        o_ref[0] = y.astype(o_ref.dtype)

    return body


def _conv_stage(xs, ws_raw, scale, shift, *, gelu, pool, th_pref=32):
    N, H, W, _ = xs[0].shape
    cins = tuple(int(a.shape[-1]) for a in xs)
    Cout = int(ws_raw[0].shape[-1])
    n_in = len(xs)
    TH = _pick_th(H, th_pref, even=pool)
    n_tiles = H // TH
    Ho, Wo = (H // 2, W // 2) if pool else (H, W)
    THo = TH // 2 if pool else TH

    if cins == (1,):
        wcat = ws_raw[0].reshape(9, Cout).astype(jnp.bfloat16)[None]
        w_spec = pl.BlockSpec((1, 9, Cout), lambda n, i: (0, 0, 0))
    else:
        K = 3 * sum(cins)
        wcat = jnp.stack([
            jnp.concatenate([ws_raw[k].astype(jnp.bfloat16)[kh, kw]
                             for k in range(n_in) for kw in range(3)], axis=0)
            for kh in range(3)])
        w_spec = pl.BlockSpec((3, K, Cout), lambda n, i: (0, 0, 0))

    return pl.pallas_call(
        _make_conv_body(TH, W, cins, Cout, n_tiles, gelu, pool),
        out_shape=jax.ShapeDtypeStruct((N, Ho, Wo, Cout), jnp.bfloat16),
        grid=(N, n_tiles),
        in_specs=([pl.BlockSpec(memory_space=pl.ANY)] * n_in + [w_spec]
                  + [pl.BlockSpec((1, Cout), lambda n, i: (0, 0)),
                     pl.BlockSpec((1, Cout), lambda n, i: (0, 0))]),
        out_specs=pl.BlockSpec((1, THo, Wo, Cout), lambda n, i: (n, i, 0, 0)),
        scratch_shapes=([pltpu.VMEM((TH + 2, W + 32, c), x.dtype)
                         for c, x in zip(cins, xs)]
                        + [pltpu.SemaphoreType.DMA((n_in, 3))]),
        compiler_params=pltpu.CompilerParams(
            dimension_semantics=("parallel", "parallel"),
            vmem_limit_bytes=_VMEM_CAP),
    )(*xs, wcat, jnp.asarray(scale, jnp.float32),
      jnp.asarray(shift, jnp.float32))


def _make_dec_body(THd, Wb, Cb, Wd, Cu, Ce, Cout, n_tiles):
    THb = THd // 2

    def body(bt_hbm, e_hbm, wu, bu, wcat, s_ref, t_ref, o_ref,
             btbuf, ubuf, ebuf, sem):
        n = pl.program_id(0)
        i = pl.program_id(1)
        r0 = i * THd
        rb0 = i * THb

        _start_slab(bt_hbm, n, rb0, THb, Wb, btbuf,
                    (sem.at[0], sem.at[1], sem.at[2]), i, n_tiles, False)
        _start_slab(e_hbm, n, r0, THd, Wd, ebuf,
                    (sem.at[3], sem.at[4], sem.at[5]), i, n_tiles, True)
        _zero_pad(ebuf, THd, Wd, Ce, i, n_tiles)

        zc = jnp.zeros((THd + 2, 16, Cu), ubuf.dtype)
        ubuf[:, 0:16, :] = zc
        ubuf[:, Wd + 16:Wd + 32, :] = zc

        _wait_slab(bt_hbm, n, THb, Wb, btbuf,
                   (sem.at[0], sem.at[1], sem.at[2]), i, n_tiles, False)

        bias = bu[...]

        def up_row(src, dh):
            a = _gelu_erf(jnp.dot(src, wu[2 * dh + 0],
                                  preferred_element_type=jnp.float32) + bias)
            b = _gelu_erf(jnp.dot(src, wu[2 * dh + 1],
                                  preferred_element_type=jnp.float32) + bias)
            return _ilv(a.astype(ubuf.dtype).reshape(-1, Wb, Cu),
                        b.astype(ubuf.dtype).reshape(-1, Wb, Cu), 1)

        xm = btbuf[1:THb + 1].reshape(THb * Wb, Cb)
        top = up_row(xm, 0)
        bot = up_row(xm, 1)
        ubuf[1:THd + 1, 16:Wd + 16] = _ilv(top, bot, 0)

        zrow = jnp.zeros((1, Wd + 32, Cu), ubuf.dtype)
        if n_tiles > 1:
            @pl.when(i > 0)
            def _():
                ubuf[0:1, 16:Wd + 16] = up_row(btbuf[0], 1)

            @pl.when(i == 0)
            def _():
                ubuf[0:1] = zrow

            @pl.when(i < n_tiles - 1)
            def _():
                ubuf[THd + 1:THd + 2, 16:Wd + 16] = up_row(btbuf[THb + 1], 0)

            @pl.when(i == n_tiles - 1)
            def _():
                ubuf[THd + 1:THd + 2] = zrow
        else:
            ubuf[0:1] = zrow
            ubuf[THd + 1:THd + 2] = zrow

        _wait_slab(e_hbm, n, THd, Wd, ebuf,
                   (sem.at[3], sem.at[4], sem.at[5]), i, n_tiles, True)

        y = _conv_acc((ubuf, ebuf), wcat, (Cu, Ce), THd, Wd)
        y = y * s_ref[...] + t_ref[...]
        o_ref[0] = y.reshape(THd, Wd, Cout).astype(o_ref.dtype)

    return body


def _dec_stage(xin, eskip, wu_raw, bu, wc_raw, scale, shift, *, thd_pref=32):
    N, Hb, Wb, Cb = xin.shape
    _, Hd, Wd, Ce = eskip.shape
    Cu = int(wu_raw.shape[-1])
    Cout = int(wc_raw.shape[-1])
    THd = _pick_th(Hd, thd_pref, even=True)
    n_tiles = Hd // THd

    wu = wu_raw.astype(jnp.bfloat16).reshape(4, Cb, Cu)
    wcb = wc_raw.astype(jnp.bfloat16)
    wcat = jnp.stack([
        jnp.concatenate([wcb[kh, kw, :Cu] for kw in range(3)]
                        + [wcb[kh, kw, Cu:] for kw in range(3)], axis=0)
        for kh in range(3)])

    return pl.pallas_call(
        _make_dec_body(THd, Wb, Cb, Wd, Cu, Ce, Cout, n_tiles),
        out_shape=jax.ShapeDtypeStruct((N, Hd, Wd, Cout), jnp.bfloat16),
        grid=(N, n_tiles),
        in_specs=[
            pl.BlockSpec(memory_space=pl.ANY),
            pl.BlockSpec(memory_space=pl.ANY),
            pl.BlockSpec((4, Cb, Cu), lambda n, i: (0, 0, 0)),
            pl.BlockSpec((1, Cu), lambda n, i: (0, 0)),
            pl.BlockSpec((3, 3 * (Cu + Ce), Cout), lambda n, i: (0, 0, 0)),
            pl.BlockSpec((1, Cout), lambda n, i: (0, 0)),
            pl.BlockSpec((1, Cout), lambda n, i: (0, 0)),
        ],
        out_specs=pl.BlockSpec((1, THd, Wd, Cout), lambda n, i: (n, i, 0, 0)),
        scratch_shapes=[
            pltpu.VMEM((THd // 2 + 2, Wb, Cb), jnp.bfloat16),
            pltpu.VMEM((THd + 2, Wd + 32, Cu), jnp.bfloat16),
            pltpu.VMEM((THd + 2, Wd + 32, Ce), jnp.bfloat16),
            pltpu.SemaphoreType.DMA((6,)),
        ],
        compiler_params=pltpu.CompilerParams(
            dimension_semantics=("parallel", "parallel"),
            vmem_limit_bytes=_VMEM_CAP),
    )(xin, eskip, wu, jnp.asarray(bu, jnp.float32), wcat,
      jnp.asarray(scale, jnp.float32), jnp.asarray(shift, jnp.float32))


def _make_head_body(THp, W2, C2, Cu):
    def body(x_ref, wu_ref, bu_ref, wp_ref, bp_ref, o_ref):
        xm = x_ref[0].reshape(THp * W2, C2)
        bias = bu_ref[...]
        wpv = wp_ref[...]
        for dh in range(2):
            for dw in range(2):
                y = _gelu_erf(jnp.dot(xm, wu_ref[2 * dh + dw],
                                      preferred_element_type=jnp.float32)
                              + bias)
                p = jnp.sum(y * wpv, axis=-1, keepdims=True) + bp_ref[...]
                o_ref[0, 2 * dh + dw] = p.reshape(THp, W2).astype(o_ref.dtype)

    return body


def _head_stage(x, wu_raw, bu, wp_raw, bp, *, thp_pref=32):
    N, H2, W2, C2 = x.shape
    Cu = int(wu_raw.shape[-1])
    THp = _pick_th(H2, thp_pref)
    n_tiles = H2 // THp

    wu = wu_raw.astype(jnp.bfloat16).reshape(4, C2, Cu)
    wp = wp_raw.reshape(1, Cu).astype(jnp.float32)

    planar = pl.pallas_call(
        _make_head_body(THp, W2, C2, Cu),
        out_shape=jax.ShapeDtypeStruct((N, 4, H2, W2), jnp.float32),
        grid=(N, n_tiles),
        in_specs=[
            pl.BlockSpec((1, THp, W2, C2), lambda n, i: (n, i, 0, 0)),
            pl.BlockSpec((4, C2, Cu), lambda n, i: (0, 0, 0)),
            pl.BlockSpec((1, Cu), lambda n, i: (0, 0)),
            pl.BlockSpec((1, Cu), lambda n, i: (0, 0)),
            pl.BlockSpec((1, 1), lambda n, i: (0, 0)),
        ],
        out_specs=pl.BlockSpec((1, 4, THp, W2), lambda n, i: (n, 0, i, 0)),
        compiler_params=pltpu.CompilerParams(
            dimension_semantics=("parallel", "parallel"),
            vmem_limit_bytes=_VMEM_CAP),
    )(x, wu, jnp.asarray(bu, jnp.float32), wp, jnp.asarray(bp, jnp.float32))
    out = planar.reshape(N, 2, 2, H2, W2).transpose(0, 3, 1, 4, 2)
    return out.reshape(N, 1, 2 * H2, 2 * W2)


def kernel(x,
           enc1__0, enc1__1, enc1__2,
           enc2__0, enc2__1, enc2__2,
           enc3__0, enc3__1, enc3__2,
           bneck1__0, bneck1__1, bneck1__2,
           bneck2__0, bneck2__1, bneck2__2,
           up3__0, up3__1,
           dec3__0, dec3__1, dec3__2,
           up2__0, up2__1,
           dec2__0, dec2__1, dec2__2,
           up1__0, up1__1,
           dec1__0, dec1__1):
    N = x.shape[0]
    F, T = x.shape[-2:]
    xh = x.reshape(N, F, T, 1)

    e1 = _conv_stage([xh], [enc1__0], enc1__1, enc1__2, gelu=True, pool=True)
    e2 = _conv_stage([e1], [enc2__0], enc2__1, enc2__2, gelu=True, pool=True)
    e3 = _conv_stage([e2], [enc3__0], enc3__1, enc3__2, gelu=True, pool=True)

    bt = _conv_stage([e3], [bneck1__0], bneck1__1, bneck1__2,
                     gelu=True, pool=False)
    bt = _conv_stage([bt], [bneck2__0], bneck2__1, bneck2__2,
                     gelu=True, pool=False)

    d3 = _dec_stage(bt, e2, up3__0, up3__1, dec3__0, dec3__1, dec3__2)
    d2 = _dec_stage(d3, e1, up2__0, up2__1, dec2__0, dec2__1, dec2__2)
    return _head_stage(d2, up1__0, up1__1, dec1__0, dec1__1)


# speedup vs baseline: 2.2454x; 1.0271x over previous
import jax
import jax.numpy as jnp
from jax.experimental import pallas as pl
from jax.experimental.pallas import tpu as pltpu

_VMEM_CAP = 48 * 1024 * 1024
_INV_SQRT2 = 0.7071067811865476


def _gelu_erf(x):
    c = 0.7978845608028654
    return 0.5 * x * (1.0 + jnp.tanh(c * x * (1.0 + 0.044715 * x * x)))


def _pick_th(H, pref, even=False):
    for th in range(min(pref, H), 0, -1):
        if H % th == 0 and (not even or th % 2 == 0):
            return th
    return H


def _ilv(a, b, axis):
    st = jnp.stack([a, b], axis=axis + 1)
    shp = list(a.shape)
    shp[axis] *= 2
    return st.reshape(shp)


def _start_slab(x, n, r0, TH, W, buf, sem3, i, n_tiles, pad_w):
    def dst(row0, nrows):
        if pad_w:
            return buf.at[pl.ds(row0, nrows), pl.ds(16, W)]
        return buf.at[pl.ds(row0, nrows)]

    pltpu.make_async_copy(x.at[n, pl.ds(r0, TH)], dst(1, TH), sem3[0]).start()
    if n_tiles > 1:
        @pl.when(i > 0)
        def _():
            pltpu.make_async_copy(
                x.at[n, pl.ds(r0 - 1, 1)], dst(0, 1), sem3[1]).start()

        @pl.when(i < n_tiles - 1)
        def _():
            pltpu.make_async_copy(
                x.at[n, pl.ds(r0 + TH, 1)], dst(TH + 1, 1), sem3[2]).start()


def _wait_slab(x, n, TH, W, buf, sem3, i, n_tiles, pad_w):
    def dst(row0, nrows):
        if pad_w:
            return buf.at[pl.ds(row0, nrows), pl.ds(16, W)]
        return buf.at[pl.ds(row0, nrows)]

    pltpu.make_async_copy(x.at[n, pl.ds(0, TH)], dst(1, TH), sem3[0]).wait()
    if n_tiles > 1:
        @pl.when(i > 0)
        def _():
            pltpu.make_async_copy(x.at[n, pl.ds(0, 1)], dst(0, 1),
                                  sem3[1]).wait()

        @pl.when(i < n_tiles - 1)
        def _():
            pltpu.make_async_copy(x.at[n, pl.ds(0, 1)], dst(TH + 1, 1),
                                  sem3[2]).wait()


def _zero_pad(buf, TH, W, c, i, n_tiles):
    zc = jnp.zeros((TH + 2, 16, c), buf.dtype)
    buf[:, 0:16, :] = zc
    buf[:, W + 16:W + 32, :] = zc
    zr = jnp.zeros((1, W + 32, c), buf.dtype)
    if n_tiles > 1:
        @pl.when(i == 0)
        def _():
            buf[0:1] = zr

        @pl.when(i == n_tiles - 1)
        def _():
            buf[TH + 1:TH + 2] = zr
    else:
        buf[0:1] = zr
        buf[TH + 1:TH + 2] = zr


def _conv_acc(bufs, wcat, cins, TH, W):
    if cins == (1,):
        cat = jnp.concatenate(
            [bufs[0][kh:kh + TH, kw + 15:kw + 15 + W, :]
             for kh in range(3) for kw in range(3)], axis=-1)
        return jnp.dot(cat.reshape(TH * W, 9), wcat[0],
                       preferred_element_type=jnp.float32)
    cat = jnp.concatenate(
        [bufs[k][:, kw + 15:kw + 15 + W, :]
         for k in range(len(cins)) for kw in range(3)], axis=-1)
    K = 3 * sum(cins)
    acc = None
    for kh in range(3):
        patch = cat[kh:kh + TH].reshape(TH * W, K)
        d = jnp.dot(patch, wcat[kh], preferred_element_type=jnp.float32)
        acc = d if acc is None else acc + d
    return acc


def _make_conv_body(TH, W, cins, Cout, n_tiles, do_gelu, do_pool):
    n_in = len(cins)

    def body(*refs):
        xs = refs[:n_in]
        wcat = refs[n_in]
        s_ref = refs[n_in + 1]
        t_ref = refs[n_in + 2]
        o_ref = refs[n_in + 3]
        bufs = refs[n_in + 4:2 * n_in + 4]
        sem = refs[2 * n_in + 4]

        n = pl.program_id(0)
        i = pl.program_id(1)
        r0 = i * TH

        for k in range(n_in):
            _start_slab(xs[k], n, r0, TH, W, bufs[k],
                        (sem.at[k, 0], sem.at[k, 1], sem.at[k, 2]),
                        i, n_tiles, True)
        for k in range(n_in):
            _zero_pad(bufs[k], TH, W, cins[k], i, n_tiles)
        for k in range(n_in):
            _wait_slab(xs[k], n, TH, W, bufs[k],
                       (sem.at[k, 0], sem.at[k, 1], sem.at[k, 2]),
                       i, n_tiles, True)

        y = _conv_acc(bufs, wcat, cins, TH, W)
        y = y * s_ref[...] + t_ref[...]
        if do_gelu:
            y = _gelu_erf(y)
        y = y.reshape(TH, W, Cout)
        if do_pool:
            y = y.reshape(TH // 2, 2, W, Cout).max(axis=1)
            y = y.reshape(TH // 2, W // 2, 2, Cout).max(axis=2)
        o_ref[0] = y.astype(o_ref.dtype)

    return body


def _conv_stage(xs, ws_raw, scale, shift, *, gelu, pool, th_pref=32):
    N, H, W, _ = xs[0].shape
    cins = tuple(int(a.shape[-1]) for a in xs)
    Cout = int(ws_raw[0].shape[-1])
    n_in = len(xs)
    TH = _pick_th(H, th_pref, even=pool)
    n_tiles = H // TH
    Ho, Wo = (H // 2, W // 2) if pool else (H, W)
    THo = TH // 2 if pool else TH

    if cins == (1,):
        wcat = ws_raw[0].reshape(9, Cout).astype(jnp.bfloat16)[None]
        w_spec = pl.BlockSpec((1, 9, Cout), lambda n, i: (0, 0, 0))
    else:
        K = 3 * sum(cins)
        wcat = jnp.stack([
            jnp.concatenate([ws_raw[k].astype(jnp.bfloat16)[kh, kw]
                             for k in range(n_in) for kw in range(3)], axis=0)
            for kh in range(3)])
        w_spec = pl.BlockSpec((3, K, Cout), lambda n, i: (0, 0, 0))

    return pl.pallas_call(
        _make_conv_body(TH, W, cins, Cout, n_tiles, gelu, pool),
        out_shape=jax.ShapeDtypeStruct((N, Ho, Wo, Cout), jnp.bfloat16),
        grid=(N, n_tiles),
        in_specs=([pl.BlockSpec(memory_space=pl.ANY)] * n_in + [w_spec]
                  + [pl.BlockSpec((1, Cout), lambda n, i: (0, 0)),
                     pl.BlockSpec((1, Cout), lambda n, i: (0, 0))]),
        out_specs=pl.BlockSpec((1, THo, Wo, Cout), lambda n, i: (n, i, 0, 0)),
        scratch_shapes=([pltpu.VMEM((TH + 2, W + 32, c), x.dtype)
                         for c, x in zip(cins, xs)]
                        + [pltpu.SemaphoreType.DMA((n_in, 3))]),
        compiler_params=pltpu.CompilerParams(
            dimension_semantics=("parallel", "parallel"),
            vmem_limit_bytes=_VMEM_CAP),
    )(*xs, wcat, jnp.asarray(scale, jnp.float32),
      jnp.asarray(shift, jnp.float32))


def _make_dec_body(THd, Wb, Cb, Wd, Cu, Ce, Cout, n_tiles):
    THb = THd // 2

    def body(bt_hbm, e_hbm, wu, bu, wcat, s_ref, t_ref, o_ref,
             btbuf, ubuf, ebuf, sem):
        n = pl.program_id(0)
        i = pl.program_id(1)
        r0 = i * THd
        rb0 = i * THb

        _start_slab(bt_hbm, n, rb0, THb, Wb, btbuf,
                    (sem.at[0], sem.at[1], sem.at[2]), i, n_tiles, False)
        _start_slab(e_hbm, n, r0, THd, Wd, ebuf,
                    (sem.at[3], sem.at[4], sem.at[5]), i, n_tiles, True)
        _zero_pad(ebuf, THd, Wd, Ce, i, n_tiles)

        zc = jnp.zeros((THd + 2, 16, Cu), ubuf.dtype)
        ubuf[:, 0:16, :] = zc
        ubuf[:, Wd + 16:Wd + 32, :] = zc

        _wait_slab(bt_hbm, n, THb, Wb, btbuf,
                   (sem.at[0], sem.at[1], sem.at[2]), i, n_tiles, False)

        bias = bu[...]

        def up_row(src, dh):
            a = _gelu_erf(jnp.dot(src, wu[2 * dh + 0],
                                  preferred_element_type=jnp.float32) + bias)
            b = _gelu_erf(jnp.dot(src, wu[2 * dh + 1],
                                  preferred_element_type=jnp.float32) + bias)
            return _ilv(a.astype(ubuf.dtype).reshape(-1, Wb, Cu),
                        b.astype(ubuf.dtype).reshape(-1, Wb, Cu), 1)

        xm = btbuf[1:THb + 1].reshape(THb * Wb, Cb)
        top = up_row(xm, 0)
        bot = up_row(xm, 1)
        ubuf[1:THd + 1, 16:Wd + 16] = _ilv(top, bot, 0)

        zrow = jnp.zeros((1, Wd + 32, Cu), ubuf.dtype)
        if n_tiles > 1:
            @pl.when(i > 0)
            def _():
                ubuf[0:1, 16:Wd + 16] = up_row(btbuf[0], 1)

            @pl.when(i == 0)
            def _():
                ubuf[0:1] = zrow

            @pl.when(i < n_tiles - 1)
            def _():
                ubuf[THd + 1:THd + 2, 16:Wd + 16] = up_row(btbuf[THb + 1], 0)

            @pl.when(i == n_tiles - 1)
            def _():
                ubuf[THd + 1:THd + 2] = zrow
        else:
            ubuf[0:1] = zrow
            ubuf[THd + 1:THd + 2] = zrow

        _wait_slab(e_hbm, n, THd, Wd, ebuf,
                   (sem.at[3], sem.at[4], sem.at[5]), i, n_tiles, True)

        y = _conv_acc((ubuf, ebuf), wcat, (Cu, Ce), THd, Wd)
        y = y * s_ref[...] + t_ref[...]
        o_ref[0] = y.reshape(THd, Wd, Cout).astype(o_ref.dtype)

    return body


def _dec_stage(xin, eskip, wu_raw, bu, wc_raw, scale, shift, *, thd_pref=32):
    N, Hb, Wb, Cb = xin.shape
    _, Hd, Wd, Ce = eskip.shape
    Cu = int(wu_raw.shape[-1])
    Cout = int(wc_raw.shape[-1])
    THd = _pick_th(Hd, thd_pref, even=True)
    n_tiles = Hd // THd

    wu = wu_raw.astype(jnp.bfloat16).reshape(4, Cb, Cu)
    wcb = wc_raw.astype(jnp.bfloat16)
    wcat = jnp.stack([
        jnp.concatenate([wcb[kh, kw, :Cu] for kw in range(3)]
                        + [wcb[kh, kw, Cu:] for kw in range(3)], axis=0)
        for kh in range(3)])

    return pl.pallas_call(
        _make_dec_body(THd, Wb, Cb, Wd, Cu, Ce, Cout, n_tiles),
        out_shape=jax.ShapeDtypeStruct((N, Hd, Wd, Cout), jnp.bfloat16),
        grid=(N, n_tiles),
        in_specs=[
            pl.BlockSpec(memory_space=pl.ANY),
            pl.BlockSpec(memory_space=pl.ANY),
            pl.BlockSpec((4, Cb, Cu), lambda n, i: (0, 0, 0)),
            pl.BlockSpec((1, Cu), lambda n, i: (0, 0)),
            pl.BlockSpec((3, 3 * (Cu + Ce), Cout), lambda n, i: (0, 0, 0)),
            pl.BlockSpec((1, Cout), lambda n, i: (0, 0)),
            pl.BlockSpec((1, Cout), lambda n, i: (0, 0)),
        ],
        out_specs=pl.BlockSpec((1, THd, Wd, Cout), lambda n, i: (n, i, 0, 0)),
        scratch_shapes=[
            pltpu.VMEM((THd // 2 + 2, Wb, Cb), jnp.bfloat16),
            pltpu.VMEM((THd + 2, Wd + 32, Cu), jnp.bfloat16),
            pltpu.VMEM((THd + 2, Wd + 32, Ce), jnp.bfloat16),
            pltpu.SemaphoreType.DMA((6,)),
        ],
        compiler_params=pltpu.CompilerParams(
            dimension_semantics=("parallel", "parallel"),
            vmem_limit_bytes=_VMEM_CAP),
    )(xin, eskip, wu, jnp.asarray(bu, jnp.float32), wcat,
      jnp.asarray(scale, jnp.float32), jnp.asarray(shift, jnp.float32))


def _make_head_body(THp, W2, C2, Cu):
    def body(x_ref, wu_ref, bu_ref, wp_ref, bp_ref, o_ref):
        xm = x_ref[0].reshape(THp * W2, C2)
        bias = bu_ref[...]
        wpv = wp_ref[...]
        for dh in range(2):
            for dw in range(2):
                y = _gelu_erf(jnp.dot(xm, wu_ref[2 * dh + dw],
                                      preferred_element_type=jnp.float32)
                              + bias)
                p = jnp.sum(y * wpv, axis=-1, keepdims=True) + bp_ref[...]
                o_ref[0, 2 * dh + dw] = p.reshape(THp, W2).astype(o_ref.dtype)

    return body


def _head_stage(x, wu_raw, bu, wp_raw, bp, *, thp_pref=32):
    N, H2, W2, C2 = x.shape
    Cu = int(wu_raw.shape[-1])
    THp = _pick_th(H2, thp_pref)
    n_tiles = H2 // THp

    wu = wu_raw.astype(jnp.bfloat16).reshape(4, C2, Cu)
    wp = wp_raw.reshape(1, Cu).astype(jnp.float32)

    planar = pl.pallas_call(
        _make_head_body(THp, W2, C2, Cu),
        out_shape=jax.ShapeDtypeStruct((N, 4, H2, W2), jnp.float32),
        grid=(N, n_tiles),
        in_specs=[
            pl.BlockSpec((1, THp, W2, C2), lambda n, i: (n, i, 0, 0)),
            pl.BlockSpec((4, C2, Cu), lambda n, i: (0, 0, 0)),
            pl.BlockSpec((1, Cu), lambda n, i: (0, 0)),
            pl.BlockSpec((1, Cu), lambda n, i: (0, 0)),
            pl.BlockSpec((1, 1), lambda n, i: (0, 0)),
        ],
        out_specs=pl.BlockSpec((1, 4, THp, W2), lambda n, i: (n, 0, i, 0)),
        compiler_params=pltpu.CompilerParams(
            dimension_semantics=("parallel", "parallel"),
            vmem_limit_bytes=_VMEM_CAP),
    )(x, wu, jnp.asarray(bu, jnp.float32), wp, jnp.asarray(bp, jnp.float32))
    out = planar.reshape(N, 2, 2, H2, W2).transpose(0, 3, 1, 4, 2)
    return out.reshape(N, 1, 2 * H2, 2 * W2)


def kernel(x,
           enc1__0, enc1__1, enc1__2,
           enc2__0, enc2__1, enc2__2,
           enc3__0, enc3__1, enc3__2,
           bneck1__0, bneck1__1, bneck1__2,
           bneck2__0, bneck2__1, bneck2__2,
           up3__0, up3__1,
           dec3__0, dec3__1, dec3__2,
           up2__0, up2__1,
           dec2__0, dec2__1, dec2__2,
           up1__0, up1__1,
           dec1__0, dec1__1):
    N = x.shape[0]
    F, T = x.shape[-2:]
    xh = x.astype(jnp.bfloat16).reshape(N, F, T, 1)

    e1 = _conv_stage([xh], [enc1__0], enc1__1, enc1__2, gelu=True, pool=True)
    e2 = _conv_stage([e1], [enc2__0], enc2__1, enc2__2, gelu=True, pool=True)
    e3 = _conv_stage([e2], [enc3__0], enc3__1, enc3__2, gelu=True, pool=True)

    bt = _conv_stage([e3], [bneck1__0], bneck1__1, bneck1__2,
                     gelu=True, pool=False)
    bt = _conv_stage([bt], [bneck2__0], bneck2__1, bneck2__2,
                     gelu=True, pool=False)

    d3 = _dec_stage(bt, e2, up3__0, up3__1, dec3__0, dec3__1, dec3__2)
    d2 = _dec_stage(d3, e1, up2__0, up2__1, dec2__0, dec2__1, dec2__2)
    return _head_stage(d2, up1__0, up1__1, dec1__0, dec1__1)
